# baseline (speedup 1.0000x reference)
import functools

import jax
import jax.numpy as jnp
from jax.experimental import pallas as pl


def _proj_kernel(ns_ref, wq_ref, wk_ref, wv_ref, q_ref, k_ref, vb_ref):
    ns = ns_ref[...]
    q_ref[...] = jax.lax.dot_general(
        ns, wq_ref[...], (((1,), (1,)), ((), ())),
        preferred_element_type=jnp.float32)
    k_ref[...] = jax.lax.dot_general(
        ns, wk_ref[...], (((1,), (1,)), ((), ())),
        preferred_element_type=jnp.float32)
    v = jax.lax.dot_general(
        ns, wv_ref[...], (((1,), (1,)), ((), ())),
        preferred_element_type=jnp.float32)
    vb_ref[...] = v.astype(jnp.bfloat16)


def _attn_kernel(q_ref, ns_ref, hid_ref, adjt_ref, k_ref, vb_ref,
                 w1_ref, b1_ref, w2_ref, b2_ref, step_ref, out_ref):
    q = q_ref[...]
    k = k_ref[...]
    adjt = adjt_ref[...]

    s = jax.lax.dot_general(
        q, k, (((1,), (1,)), ((), ())), preferred_element_type=jnp.float32)
    neg = jnp.float32(-1e30)
    m = jnp.max(jnp.where(adjt, s, neg), axis=1, keepdims=True)
    e = jnp.where(adjt, jnp.exp2(s - m), 0.0).astype(jnp.bfloat16)
    denom = jnp.sum(e, axis=1, keepdims=True, dtype=jnp.float32)
    accn = jax.lax.dot_general(
        e, vb_ref[...], (((1,), (0,)), ((), ())),
        preferred_element_type=jnp.float32)
    acc = accn * (1.0 / denom)

    nps = jnp.concatenate([ns_ref[...], acc], axis=1)
    h = jax.lax.dot_general(
        nps, w1_ref[...], (((1,), (1,)), ((), ())),
        preferred_element_type=jnp.float32) + b1_ref[...]
    h = jnp.maximum(h, 0.0)
    upd = jax.lax.dot_general(
        h, w2_ref[...], (((1,), (1,)), ((), ())),
        preferred_element_type=jnp.float32) + b2_ref[...]
    out_ref[...] = hid_ref[...] + step_ref[0, 0] * upd


@functools.partial(jax.jit, static_argnames=())
def kernel(input_states, hidden_states, adjacency_matrix, Wq, Wk, Wv,
           W1, b1, W2, b2, step_size):
    n, in_sz = input_states.shape
    hid_sz = hidden_states.shape[1]
    d = in_sz + hid_sz
    mlp_h = W1.shape[0]

    ns = jnp.concatenate([input_states, hidden_states], axis=1)
    Wq = Wq * jnp.float32(1.4426950408889634)
    adjt = adjacency_matrix.T

    q, k, v = pl.pallas_call(
        _proj_kernel,
        out_shape=[jax.ShapeDtypeStruct((n, d), jnp.float32),
                   jax.ShapeDtypeStruct((n, d), jnp.float32),
                   jax.ShapeDtypeStruct((n, d), jnp.bfloat16)],
    )(ns, Wq, Wk, Wv)

    r = 256
    grid = (n // r,)
    out = pl.pallas_call(
        _attn_kernel,
        grid=grid,
        in_specs=[
            pl.BlockSpec((r, d), lambda i: (i, 0)),
            pl.BlockSpec((r, d), lambda i: (i, 0)),
            pl.BlockSpec((r, hid_sz), lambda i: (i, 0)),
            pl.BlockSpec((r, n), lambda i: (i, 0)),
            pl.BlockSpec((n, d), lambda i: (0, 0)),
            pl.BlockSpec((n, d), lambda i: (0, 0)),
            pl.BlockSpec((mlp_h, 2 * d), lambda i: (0, 0)),
            pl.BlockSpec((1, mlp_h), lambda i: (0, 0)),
            pl.BlockSpec((hid_sz, mlp_h), lambda i: (0, 0)),
            pl.BlockSpec((1, hid_sz), lambda i: (0, 0)),
            pl.BlockSpec((1, 1), lambda i: (0, 0)),
        ],
        out_specs=pl.BlockSpec((r, hid_sz), lambda i: (i, 0)),
        out_shape=jax.ShapeDtypeStruct((n, hid_sz), jnp.float32),
    )(q, ns, hidden_states, adjt, k, v,
      W1, b1.reshape(1, mlp_h), W2, b2.reshape(1, hid_sz),
      step_size.reshape(1, 1))
    return out


# speedup vs baseline: 1.5247x; 1.5247x over previous
import functools

import jax
import jax.numpy as jnp
from jax.experimental import pallas as pl

_LOG2E = 1.4426950408889634
_CLAMP = 104.0


def _proj_kernel(ns_ref, wq_ref, wk_ref, wv_ref, q_ref, k_ref, vt_ref):
    ns = ns_ref[...]
    q_ref[...] = jax.lax.dot_general(
        ns, wq_ref[...], (((1,), (1,)), ((), ())),
        preferred_element_type=jnp.float32)
    k_ref[...] = jax.lax.dot_general(
        ns, wk_ref[...], (((1,), (1,)), ((), ())),
        preferred_element_type=jnp.float32)
    v = jax.lax.dot_general(
        ns, wv_ref[...], (((1,), (1,)), ((), ())),
        preferred_element_type=jnp.float32)
    vt = v.astype(jnp.bfloat16).T
    ones = jnp.ones((8, v.shape[0]), dtype=jnp.bfloat16)
    vt_ref[...] = jnp.concatenate([vt, ones], axis=0)


def _agg_kernel(k_ref, adj_ref, q_ref, vt_ref, acct_ref):
    j = pl.program_id(0)
    k_blk = k_ref[...]
    adj = adj_ref[...]
    s = jax.lax.dot_general(
        k_blk, q_ref[...], (((1,), (1,)), ((), ())),
        preferred_element_type=jnp.float32)
    e = jnp.where(adj, jnp.exp2(jnp.minimum(s, _CLAMP)), 0.0)
    e = e.astype(jnp.bfloat16)
    part = jax.lax.dot_general(
        vt_ref[...], e, (((1,), (0,)), ((), ())),
        preferred_element_type=jnp.float32)

    @pl.when(j == 0)
    def _init():
        acct_ref[...] = part

    @pl.when(j != 0)
    def _acc():
        acct_ref[...] += part


def _mlp_kernel(acct_ref, ns_ref, hid_ref, w1_ref, b1_ref, w2_ref, b2_ref,
                step_ref, out_ref):
    acct = acct_ref[...]
    d = acct.shape[0] - 8
    denom = acct[d:d + 1, :]
    accn_t = acct[:d, :] * (1.0 / denom)
    acc = accn_t.T
    nps = jnp.concatenate([ns_ref[...], acc], axis=1)
    h = jax.lax.dot_general(
        nps, w1_ref[...], (((1,), (1,)), ((), ())),
        preferred_element_type=jnp.float32) + b1_ref[...]
    h = jnp.maximum(h, 0.0)
    upd = jax.lax.dot_general(
        h, w2_ref[...], (((1,), (1,)), ((), ())),
        preferred_element_type=jnp.float32) + b2_ref[...]
    out_ref[...] = hid_ref[...] + step_ref[0, 0] * upd


@functools.partial(jax.jit, static_argnames=())
def kernel(input_states, hidden_states, adjacency_matrix, Wq, Wk, Wv,
           W1, b1, W2, b2, step_size):
    n, in_sz = input_states.shape
    hid_sz = hidden_states.shape[1]
    d = in_sz + hid_sz
    mlp_h = W1.shape[0]

    ns = jnp.concatenate([input_states, hidden_states], axis=1)
    Wq = Wq * jnp.float32(_LOG2E)

    q, k, vt = pl.pallas_call(
        _proj_kernel,
        out_shape=[jax.ShapeDtypeStruct((n, d), jnp.float32),
                   jax.ShapeDtypeStruct((n, d), jnp.float32),
                   jax.ShapeDtypeStruct((d + 8, n), jnp.bfloat16)],
    )(ns, Wq, Wk, Wv)

    r = 256
    acct = pl.pallas_call(
        _agg_kernel,
        grid=(n // r,),
        in_specs=[
            pl.BlockSpec((r, d), lambda j: (j, 0)),
            pl.BlockSpec((r, n), lambda j: (j, 0)),
            pl.BlockSpec((n, d), lambda j: (0, 0)),
            pl.BlockSpec((d + 8, r), lambda j: (0, j)),
        ],
        out_specs=pl.BlockSpec((d + 8, n), lambda j: (0, 0)),
        out_shape=jax.ShapeDtypeStruct((d + 8, n), jnp.float32),
    )(k, adjacency_matrix, q, vt)

    re = min(1024, n)
    out = pl.pallas_call(
        _mlp_kernel,
        grid=(n // re,),
        in_specs=[
            pl.BlockSpec((d + 8, re), lambda i: (0, i)),
            pl.BlockSpec((re, d), lambda i: (i, 0)),
            pl.BlockSpec((re, hid_sz), lambda i: (i, 0)),
            pl.BlockSpec((mlp_h, 2 * d), lambda i: (0, 0)),
            pl.BlockSpec((1, mlp_h), lambda i: (0, 0)),
            pl.BlockSpec((hid_sz, mlp_h), lambda i: (0, 0)),
            pl.BlockSpec((1, hid_sz), lambda i: (0, 0)),
            pl.BlockSpec((1, 1), lambda i: (0, 0)),
        ],
        out_specs=pl.BlockSpec((re, hid_sz), lambda i: (i, 0)),
        out_shape=jax.ShapeDtypeStruct((n, hid_sz), jnp.float32),
    )(acct, ns, hidden_states,
      W1, b1.reshape(1, mlp_h), W2, b2.reshape(1, hid_sz),
      step_size.reshape(1, 1))
    return out


# speedup vs baseline: 1.5389x; 1.0093x over previous
import functools

import jax
import jax.numpy as jnp
from jax.experimental import pallas as pl

_LOG2E = 1.4426950408889634
_CLAMP = 104.0


def _proj_kernel(ns_ref, wq_ref, wk_ref, wv_ref, q_ref, k_ref, vt_ref):
    ns = ns_ref[...]
    q_ref[...] = jax.lax.dot_general(
        ns, wq_ref[...], (((1,), (1,)), ((), ())),
        preferred_element_type=jnp.float32).astype(jnp.bfloat16)
    k_ref[...] = jax.lax.dot_general(
        ns, wk_ref[...], (((1,), (1,)), ((), ())),
        preferred_element_type=jnp.float32).astype(jnp.bfloat16)
    v = jax.lax.dot_general(
        ns, wv_ref[...], (((1,), (1,)), ((), ())),
        preferred_element_type=jnp.float32)
    vt = v.astype(jnp.bfloat16).T
    ones = jnp.ones((8, v.shape[0]), dtype=jnp.bfloat16)
    vt_ref[...] = jnp.concatenate([vt, ones], axis=0)


def _agg_kernel(k_ref, adj_ref, q_ref, vt_ref, acct_ref):
    j = pl.program_id(0)
    k_blk = k_ref[...]
    adj = adj_ref[...]
    s = jax.lax.dot_general(
        k_blk, q_ref[...], (((1,), (1,)), ((), ())),
        preferred_element_type=jnp.float32)
    e = jnp.where(adj, jnp.exp2(jnp.minimum(s, _CLAMP)), 0.0)
    e = e.astype(jnp.bfloat16)
    part = jax.lax.dot_general(
        vt_ref[...], e, (((1,), (0,)), ((), ())),
        preferred_element_type=jnp.float32)

    @pl.when(j == 0)
    def _init():
        acct_ref[...] = part

    @pl.when(j != 0)
    def _acc():
        acct_ref[...] += part


def _mlp_kernel(acct_ref, ns_ref, hid_ref, w1_ref, b1_ref, w2_ref, b2_ref,
                step_ref, out_ref):
    acct = acct_ref[...]
    d = acct.shape[0] - 8
    denom = acct[d:d + 1, :]
    accn_t = acct[:d, :] * (1.0 / denom)
    acc = accn_t.T
    nps = jnp.concatenate([ns_ref[...], acc], axis=1)
    h = jax.lax.dot_general(
        nps, w1_ref[...], (((1,), (1,)), ((), ())),
        preferred_element_type=jnp.float32) + b1_ref[...]
    h = jnp.maximum(h, 0.0)
    upd = jax.lax.dot_general(
        h, w2_ref[...], (((1,), (1,)), ((), ())),
        preferred_element_type=jnp.float32) + b2_ref[...]
    out_ref[...] = hid_ref[...] + step_ref[0, 0] * upd


@functools.partial(jax.jit, static_argnames=())
def kernel(input_states, hidden_states, adjacency_matrix, Wq, Wk, Wv,
           W1, b1, W2, b2, step_size):
    n, in_sz = input_states.shape
    hid_sz = hidden_states.shape[1]
    d = in_sz + hid_sz
    mlp_h = W1.shape[0]

    ns = jnp.concatenate([input_states, hidden_states], axis=1)
    Wq = Wq * jnp.float32(_LOG2E)

    q, k, vt = pl.pallas_call(
        _proj_kernel,
        out_shape=[jax.ShapeDtypeStruct((n, d), jnp.bfloat16),
                   jax.ShapeDtypeStruct((n, d), jnp.bfloat16),
                   jax.ShapeDtypeStruct((d + 8, n), jnp.bfloat16)],
    )(ns, Wq, Wk, Wv)

    r = 256
    acct = pl.pallas_call(
        _agg_kernel,
        grid=(n // r,),
        in_specs=[
            pl.BlockSpec((r, d), lambda j: (j, 0)),
            pl.BlockSpec((r, n), lambda j: (j, 0)),
            pl.BlockSpec((n, d), lambda j: (0, 0)),
            pl.BlockSpec((d + 8, r), lambda j: (0, j)),
        ],
        out_specs=pl.BlockSpec((d + 8, n), lambda j: (0, 0)),
        out_shape=jax.ShapeDtypeStruct((d + 8, n), jnp.float32),
    )(k, adjacency_matrix, q, vt)

    re = min(1024, n)
    out = pl.pallas_call(
        _mlp_kernel,
        grid=(n // re,),
        in_specs=[
            pl.BlockSpec((d + 8, re), lambda i: (0, i)),
            pl.BlockSpec((re, d), lambda i: (i, 0)),
            pl.BlockSpec((re, hid_sz), lambda i: (i, 0)),
            pl.BlockSpec((mlp_h, 2 * d), lambda i: (0, 0)),
            pl.BlockSpec((1, mlp_h), lambda i: (0, 0)),
            pl.BlockSpec((hid_sz, mlp_h), lambda i: (0, 0)),
            pl.BlockSpec((1, hid_sz), lambda i: (0, 0)),
            pl.BlockSpec((1, 1), lambda i: (0, 0)),
        ],
        out_specs=pl.BlockSpec((re, hid_sz), lambda i: (i, 0)),
        out_shape=jax.ShapeDtypeStruct((n, hid_sz), jnp.float32),
    )(acct, ns, hidden_states,
      W1, b1.reshape(1, mlp_h), W2, b2.reshape(1, hid_sz),
      step_size.reshape(1, 1))
    return out


# speedup vs baseline: 1.6035x; 1.0420x over previous
import functools

import jax
import jax.numpy as jnp
from jax.experimental import pallas as pl

_LOG2E = 1.4426950408889634
_CLAMP = 104.0


def _proj_kernel(ns_ref, wq_ref, wk_ref, wv_ref, q_ref, k_ref, vt_ref):
    ns = ns_ref[...]
    q_ref[...] = jax.lax.dot_general(
        ns, wq_ref[...], (((1,), (1,)), ((), ())),
        preferred_element_type=jnp.float32).astype(jnp.bfloat16)
    k_ref[...] = jax.lax.dot_general(
        ns, wk_ref[...], (((1,), (1,)), ((), ())),
        preferred_element_type=jnp.float32).astype(jnp.bfloat16)
    v = jax.lax.dot_general(
        ns, wv_ref[...], (((1,), (1,)), ((), ())),
        preferred_element_type=jnp.float32)
    vt = v.astype(jnp.bfloat16).T
    ones = jnp.ones((8, v.shape[0]), dtype=jnp.bfloat16)
    vt_ref[...] = jnp.concatenate([vt, ones], axis=0)


def _agg_kernel(k_ref, adj_ref, q_ref, vt_ref, acct_ref):
    j = pl.program_id(0)
    k_blk = k_ref[...]
    adj = adj_ref[...]
    s = jax.lax.dot_general(
        k_blk, q_ref[...], (((1,), (1,)), ((), ())),
        preferred_element_type=jnp.float32)
    e = jnp.where(adj, jnp.exp2(jnp.minimum(s, _CLAMP)), 0.0)
    e = e.astype(jnp.bfloat16)
    part = jax.lax.dot_general(
        vt_ref[...], e, (((1,), (0,)), ((), ())),
        preferred_element_type=jnp.float32)

    @pl.when(j == 0)
    def _init():
        acct_ref[...] = part

    @pl.when(j != 0)
    def _acc():
        acct_ref[...] += part


def _mlp_kernel(acct_ref, ns_ref, hid_ref, w1_ref, b1_ref, w2_ref, b2_ref,
                step_ref, out_ref):
    acct = acct_ref[...]
    d = acct.shape[0] - 8
    denom = acct[d:d + 1, :]
    accn_t = acct[:d, :] * (1.0 / denom)
    acc = accn_t.T
    nps = jnp.concatenate([ns_ref[...], acc], axis=1)
    h = jax.lax.dot_general(
        nps, w1_ref[...], (((1,), (1,)), ((), ())),
        preferred_element_type=jnp.float32) + b1_ref[...]
    h = jnp.maximum(h, 0.0)
    upd = jax.lax.dot_general(
        h, w2_ref[...], (((1,), (1,)), ((), ())),
        preferred_element_type=jnp.float32) + b2_ref[...]
    out_ref[...] = hid_ref[...] + step_ref[0, 0] * upd


@functools.partial(jax.jit, static_argnames=())
def kernel(input_states, hidden_states, adjacency_matrix, Wq, Wk, Wv,
           W1, b1, W2, b2, step_size):
    n, in_sz = input_states.shape
    hid_sz = hidden_states.shape[1]
    d = in_sz + hid_sz
    mlp_h = W1.shape[0]

    ns = jnp.concatenate([input_states, hidden_states], axis=1)
    Wq = Wq * jnp.float32(_LOG2E)

    q, k, vt = pl.pallas_call(
        _proj_kernel,
        out_shape=[jax.ShapeDtypeStruct((n, d), jnp.bfloat16),
                   jax.ShapeDtypeStruct((n, d), jnp.bfloat16),
                   jax.ShapeDtypeStruct((d + 8, n), jnp.bfloat16)],
    )(ns, Wq, Wk, Wv)

    r = 512
    acct = pl.pallas_call(
        _agg_kernel,
        grid=(n // r,),
        in_specs=[
            pl.BlockSpec((r, d), lambda j: (j, 0)),
            pl.BlockSpec((r, n), lambda j: (j, 0)),
            pl.BlockSpec((n, d), lambda j: (0, 0)),
            pl.BlockSpec((d + 8, r), lambda j: (0, j)),
        ],
        out_specs=pl.BlockSpec((d + 8, n), lambda j: (0, 0)),
        out_shape=jax.ShapeDtypeStruct((d + 8, n), jnp.float32),
    )(k, adjacency_matrix, q, vt)

    re = min(1024, n)
    out = pl.pallas_call(
        _mlp_kernel,
        grid=(n // re,),
        in_specs=[
            pl.BlockSpec((d + 8, re), lambda i: (0, i)),
            pl.BlockSpec((re, d), lambda i: (i, 0)),
            pl.BlockSpec((re, hid_sz), lambda i: (i, 0)),
            pl.BlockSpec((mlp_h, 2 * d), lambda i: (0, 0)),
            pl.BlockSpec((1, mlp_h), lambda i: (0, 0)),
            pl.BlockSpec((hid_sz, mlp_h), lambda i: (0, 0)),
            pl.BlockSpec((1, hid_sz), lambda i: (0, 0)),
            pl.BlockSpec((1, 1), lambda i: (0, 0)),
        ],
        out_specs=pl.BlockSpec((re, hid_sz), lambda i: (i, 0)),
        out_shape=jax.ShapeDtypeStruct((n, hid_sz), jnp.float32),
    )(acct, ns, hidden_states,
      W1, b1.reshape(1, mlp_h), W2, b2.reshape(1, hid_sz),
      step_size.reshape(1, 1))
    return out


# speedup vs baseline: 2.2569x; 1.4075x over previous
import functools

import jax
import jax.numpy as jnp
from jax.experimental import pallas as pl
from jax.experimental.pallas import tpu as pltpu

_LOG2E = 1.4426950408889634
_CLAMP = 104.0


def _fused_kernel(ns_blk_ref, adj_ref, ns_ref, hid_ref, wq_ref, wk_ref,
                  wv_ref, w1_ref, b1_ref, w2_ref, b2_ref, step_ref,
                  out_ref, q_ref, acct_ref):
    j = pl.program_id(0)
    nsteps = pl.num_programs(0)

    @pl.when(j == 0)
    def _compute_q():
        q_ref[...] = jax.lax.dot_general(
            ns_ref[...], wq_ref[...], (((1,), (1,)), ((), ())),
            preferred_element_type=jnp.float32).astype(jnp.bfloat16)

    ns_blk = ns_blk_ref[...]
    k_blk = jax.lax.dot_general(
        ns_blk, wk_ref[...], (((1,), (1,)), ((), ())),
        preferred_element_type=jnp.float32).astype(jnp.bfloat16)
    v_blk = jax.lax.dot_general(
        ns_blk, wv_ref[...], (((1,), (1,)), ((), ())),
        preferred_element_type=jnp.float32).astype(jnp.bfloat16)
    ones = jnp.ones((8, v_blk.shape[0]), dtype=jnp.bfloat16)
    vt_blk = jnp.concatenate([v_blk.T, ones], axis=0)

    s = jax.lax.dot_general(
        k_blk, q_ref[...], (((1,), (1,)), ((), ())),
        preferred_element_type=jnp.float32)
    adj = adj_ref[...] != 0
    e = jnp.where(adj, jnp.exp2(jnp.minimum(s, _CLAMP)), 0.0)
    e = e.astype(jnp.bfloat16)
    part = jax.lax.dot_general(
        vt_blk, e, (((1,), (0,)), ((), ())),
        preferred_element_type=jnp.float32)

    @pl.when(j == 0)
    def _init():
        acct_ref[...] = part

    @pl.when(j != 0)
    def _acc():
        acct_ref[...] += part

    @pl.when(j == nsteps - 1)
    def _epilogue():
        acct = acct_ref[...]
        d = acct.shape[0] - 8
        denom = acct[d:d + 1, :]
        acc = (acct[:d, :] * (1.0 / denom)).T
        nps = jnp.concatenate([ns_ref[...], acc], axis=1)
        h = jax.lax.dot_general(
            nps, w1_ref[...], (((1,), (1,)), ((), ())),
            preferred_element_type=jnp.float32) + b1_ref[...]
        h = jnp.maximum(h, 0.0)
        upd = jax.lax.dot_general(
            h, w2_ref[...], (((1,), (1,)), ((), ())),
            preferred_element_type=jnp.float32) + b2_ref[...]
        out_ref[...] = hid_ref[...] + step_ref[0, 0] * upd


@functools.partial(jax.jit, static_argnames=())
def kernel(input_states, hidden_states, adjacency_matrix, Wq, Wk, Wv,
           W1, b1, W2, b2, step_size):
    n, in_sz = input_states.shape
    hid_sz = hidden_states.shape[1]
    d = in_sz + hid_sz
    mlp_h = W1.shape[0]

    ns = jnp.concatenate([input_states, hidden_states], axis=1)
    adj8 = adjacency_matrix.view(jnp.int8)
    Wq = Wq * jnp.float32(_LOG2E)

    r = min(512, n)
    out = pl.pallas_call(
        _fused_kernel,
        grid=(n // r,),
        in_specs=[
            pl.BlockSpec((r, d), lambda j: (j, 0)),
            pl.BlockSpec((r, n), lambda j: (j, 0)),
            pl.BlockSpec((n, d), lambda j: (0, 0)),
            pl.BlockSpec((n, hid_sz), lambda j: (0, 0)),
            pl.BlockSpec((d, d), lambda j: (0, 0)),
            pl.BlockSpec((d, d), lambda j: (0, 0)),
            pl.BlockSpec((d, d), lambda j: (0, 0)),
            pl.BlockSpec((mlp_h, 2 * d), lambda j: (0, 0)),
            pl.BlockSpec((1, mlp_h), lambda j: (0, 0)),
            pl.BlockSpec((hid_sz, mlp_h), lambda j: (0, 0)),
            pl.BlockSpec((1, hid_sz), lambda j: (0, 0)),
            pl.BlockSpec((1, 1), lambda j: (0, 0)),
        ],
        out_specs=pl.BlockSpec((n, hid_sz), lambda j: (0, 0)),
        out_shape=jax.ShapeDtypeStruct((n, hid_sz), jnp.float32),
        scratch_shapes=[
            pltpu.VMEM((n, d), jnp.bfloat16),
            pltpu.VMEM((d + 8, n), jnp.float32),
        ],
    )(ns, adj8, ns, hidden_states, Wq, Wk, Wv,
      W1, b1.reshape(1, mlp_h), W2, b2.reshape(1, hid_sz),
      step_size.reshape(1, 1))
    return out


# speedup vs baseline: 2.7121x; 1.2017x over previous
import functools

import jax
import jax.numpy as jnp
from jax.experimental import pallas as pl
from jax.experimental.pallas import tpu as pltpu

_LOG2E = 1.4426950408889634
_CLAMP = 104.0


def _fused_kernel(ns_blk_ref, adj_ref, ns_ref, hid_ref, wq_ref, wk_ref,
                  wv_ref, w1_ref, b1_ref, w2_ref, b2_ref, step_ref,
                  out_ref, q_ref, acct_ref):
    j = pl.program_id(0)
    nsteps = pl.num_programs(0)

    @pl.when(j == 0)
    def _compute_q():
        q_ref[...] = jax.lax.dot_general(
            ns_ref[...], wq_ref[...], (((1,), (1,)), ((), ())),
            preferred_element_type=jnp.float32).astype(jnp.bfloat16)

    ns_blk = ns_blk_ref[...]
    k_blk = jax.lax.dot_general(
        ns_blk, wk_ref[...], (((1,), (1,)), ((), ())),
        preferred_element_type=jnp.float32).astype(jnp.bfloat16)
    v_blk = jax.lax.dot_general(
        ns_blk, wv_ref[...], (((1,), (1,)), ((), ())),
        preferred_element_type=jnp.float32).astype(jnp.bfloat16)
    ones = jnp.ones((8, v_blk.shape[0]), dtype=jnp.bfloat16)
    vt_blk = jnp.concatenate([v_blk.T, ones], axis=0)

    s = jax.lax.dot_general(
        k_blk, q_ref[...], (((1,), (1,)), ((), ())),
        preferred_element_type=jnp.float32)
    adjb = adj_ref[...].astype(jnp.bfloat16)
    e = jnp.exp2(jnp.minimum(s, _CLAMP)).astype(jnp.bfloat16) * adjb
    part = jax.lax.dot_general(
        vt_blk, e, (((1,), (0,)), ((), ())),
        preferred_element_type=jnp.float32)

    @pl.when(j == 0)
    def _init():
        acct_ref[...] = part

    @pl.when(j != 0)
    def _acc():
        acct_ref[...] += part

    @pl.when(j == nsteps - 1)
    def _epilogue():
        acct = acct_ref[...]
        d = acct.shape[0] - 8
        denom = acct[d:d + 1, :]
        acc = (acct[:d, :] * (1.0 / denom)).T
        nps = jnp.concatenate([ns_ref[...], acc], axis=1)
        h = jax.lax.dot_general(
            nps, w1_ref[...], (((1,), (1,)), ((), ())),
            preferred_element_type=jnp.float32) + b1_ref[...]
        h = jnp.maximum(h, 0.0)
        upd = jax.lax.dot_general(
            h, w2_ref[...], (((1,), (1,)), ((), ())),
            preferred_element_type=jnp.float32) + b2_ref[...]
        out_ref[...] = hid_ref[...] + step_ref[0, 0] * upd


@functools.partial(jax.jit, static_argnames=())
def kernel(input_states, hidden_states, adjacency_matrix, Wq, Wk, Wv,
           W1, b1, W2, b2, step_size):
    n, in_sz = input_states.shape
    hid_sz = hidden_states.shape[1]
    d = in_sz + hid_sz
    mlp_h = W1.shape[0]

    ns = jnp.concatenate([input_states, hidden_states], axis=1)
    adj8 = adjacency_matrix.view(jnp.int8)
    Wq = Wq * jnp.float32(_LOG2E)

    r = min(512, n)
    out = pl.pallas_call(
        _fused_kernel,
        grid=(n // r,),
        in_specs=[
            pl.BlockSpec((r, d), lambda j: (j, 0)),
            pl.BlockSpec((r, n), lambda j: (j, 0)),
            pl.BlockSpec((n, d), lambda j: (0, 0)),
            pl.BlockSpec((n, hid_sz), lambda j: (0, 0)),
            pl.BlockSpec((d, d), lambda j: (0, 0)),
            pl.BlockSpec((d, d), lambda j: (0, 0)),
            pl.BlockSpec((d, d), lambda j: (0, 0)),
            pl.BlockSpec((mlp_h, 2 * d), lambda j: (0, 0)),
            pl.BlockSpec((1, mlp_h), lambda j: (0, 0)),
            pl.BlockSpec((hid_sz, mlp_h), lambda j: (0, 0)),
            pl.BlockSpec((1, hid_sz), lambda j: (0, 0)),
            pl.BlockSpec((1, 1), lambda j: (0, 0)),
        ],
        out_specs=pl.BlockSpec((n, hid_sz), lambda j: (0, 0)),
        out_shape=jax.ShapeDtypeStruct((n, hid_sz), jnp.float32),
        scratch_shapes=[
            pltpu.VMEM((n, d), jnp.bfloat16),
            pltpu.VMEM((d + 8, n), jnp.float32),
        ],
    )(ns, adj8, ns, hidden_states, Wq, Wk, Wv,
      W1, b1.reshape(1, mlp_h), W2, b2.reshape(1, hid_sz),
      step_size.reshape(1, 1))
    return out
